# Initial kernel scaffold; baseline (speedup 1.0000x reference)
import jax
import jax.numpy as jnp
from jax.experimental import pallas as pl


def kernel(x, edge_index, edge_attr, W1, b1, W2, b2, eps, gamma, beta):
    raise NotImplementedError("write your pallas kernel here")


# speedup vs baseline: 1.7798x; 1.7798x over previous
import functools

import jax
import jax.numpy as jnp
import numpy as np
from jax import lax
from jax.experimental import pallas as pl
from jax.experimental.pallas import tpu as pltpu
from jax.experimental.pallas import tpu_sc as plsc

N_NODES = 100000
N_EDGES = 1600000
HIDDEN = 50
NCHUNK = 4
CW = 16
CHUNK_COLS = ((0, 13), (13, 13), (26, 13), (39, 11))

N_SUB = 16
BLK_E = 800
EDGES_PER_SUB = N_EDGES // N_SUB
N_BLK = EDGES_PER_SUB // BLK_E
NP = 100096
ROWS_PER_SUB = NP // N_SUB
ZROWS = 136

_g_of_c = np.arange(HIDDEN) // 5
_GM = np.zeros((HIDDEN, 10), np.float32)
_GM[np.arange(HIDDEN), _g_of_c] = 0.2
_PM = (_g_of_c[None, :] == np.arange(10)[:, None]).astype(np.float32)


def _split_chunks(x, blk):
    n = x.shape[0]

    def body(x_ref, o_ref):
        x_blk = x_ref[...]
        outs = []
        for c0, w in CHUNK_COLS:
            chunk = x_blk[:, c0:c0 + w]
            outs.append(jnp.pad(chunk, ((0, 0), (0, CW - w))))
        o_ref[...] = jnp.stack(outs, axis=0)

    return pl.pallas_call(
        body,
        grid=(n // blk,),
        in_specs=[pl.BlockSpec((blk, HIDDEN), lambda i: (i, 0))],
        out_specs=pl.BlockSpec((NCHUNK, blk, CW), lambda i: (0, i, 0)),
        out_shape=jax.ShapeDtypeStruct((NCHUNK, n, CW), jnp.float32),
    )(x)


def _sc_agg(hc_flat, attr_flat, src, dst):
    mesh = plsc.VectorSubcoreMesh(core_axis_name="c", subcore_axis_name="s")

    @functools.partial(
        pl.kernel,
        out_type=jax.ShapeDtypeStruct((NCHUNK * NP, CW), jnp.float32),
        mesh=mesh,
        scratch_types=[
            pltpu.VMEM_SHARED((NP, CW), jnp.float32),
            pltpu.VMEM((BLK_E,), jnp.int32),
            pltpu.VMEM((BLK_E,), jnp.int32),
            pltpu.VMEM((BLK_E, CW), jnp.float32),
            pltpu.VMEM((BLK_E, CW), jnp.float32),
            pltpu.VMEM((ZROWS, CW), jnp.float32),
        ],
        compiler_params=pltpu.CompilerParams(use_tc_tiling_on_sc=False),
    )
    def k(hc_hbm, attr_hbm, src_hbm, dst_hbm, out_hbm,
          acc_sh, src_v, dst_v, hrow_v, attr_v, zero_v):
        c = lax.axis_index("c")
        s = lax.axis_index("s")

        zvec = jnp.zeros((CW,), jnp.float32)

        @pl.loop(0, ZROWS, unroll=8)
        def _(i):
            zero_v[i] = zvec

        for chunk_i in range(NCHUNK // 2):
            kk = 2 * c + chunk_i

            @pl.loop(0, ROWS_PER_SUB, step=ZROWS)
            def _(r):
                pltpu.sync_copy(zero_v,
                                acc_sh.at[pl.ds(s * ROWS_PER_SUB + r, ZROWS)])

            plsc.subcore_barrier()

            @pl.loop(0, N_BLK)
            def _(j):
                e0 = s * EDGES_PER_SUB + j * BLK_E
                pltpu.sync_copy(src_hbm.at[pl.ds(e0, BLK_E)], src_v)
                pltpu.sync_copy(dst_hbm.at[pl.ds(e0, BLK_E)], dst_v)
                pltpu.sync_copy(attr_hbm.at[pl.ds(kk * N_EDGES + e0, BLK_E)],
                                attr_v)

                off = kk * N_NODES

                @pl.loop(0, BLK_E, step=16, unroll=8)
                def _(i):
                    src_v[pl.ds(i, 16)] = src_v[pl.ds(i, 16)] + off

                pltpu.sync_copy(hc_hbm.at[src_v], hrow_v)

                @pl.loop(0, BLK_E, unroll=16)
                def _(r):
                    hrow_v[r] = jnp.maximum(hrow_v[r] + attr_v[r], 0.0)

                pltpu.sync_copy(hrow_v, acc_sh.at[dst_v], add=True)

            plsc.subcore_barrier()

            pltpu.sync_copy(
                acc_sh.at[pl.ds(s * ROWS_PER_SUB, ROWS_PER_SUB)],
                out_hbm.at[pl.ds(kk * NP + s * ROWS_PER_SUB,
                                 ROWS_PER_SUB)])

    return k(hc_flat, attr_flat, src, dst)


def _mlp_gn(h, agg, w1t, b1, w2t, b2, epsl, gamma, beta, last):
    blk = 2000
    gm = jnp.asarray(_GM)
    pm = jnp.asarray(_PM)

    def body(h_ref, agg_ref, w1_ref, b1_ref, w2_ref, b2_ref, gm_ref, pm_ref,
             gamma_ref, beta_ref, eps_ref, o_ref, *rest):
        h_blk = h_ref[...]
        a = agg_ref[...]
        agg_blk = jnp.concatenate(
            [a[ci, :, :w] for ci, (c0, w) in enumerate(CHUNK_COLS)], axis=1)
        z = (1.0 + eps_ref[0, 0]) * h_blk + agg_blk
        z = jnp.maximum(
            jnp.dot(z, w1_ref[...], preferred_element_type=jnp.float32)
            + b1_ref[...], 0.0)
        z = jnp.dot(z, w2_ref[...], preferred_element_type=jnp.float32) \
            + b2_ref[...]
        mean = jnp.dot(z, gm_ref[...], preferred_element_type=jnp.float32)
        zc = z - jnp.dot(mean, pm_ref[...], preferred_element_type=jnp.float32)
        var = jnp.dot(zc * zc, gm_ref[...], preferred_element_type=jnp.float32)
        rstd = lax.rsqrt(var + 1e-5)
        zn = zc * jnp.dot(rstd, pm_ref[...], preferred_element_type=jnp.float32)
        out = zn * gamma_ref[...] + beta_ref[...]
        if not last:
            out = jnp.maximum(out, 0.0)
            o_ref[...] = out
            hc_ref = rest[0]
            outs = []
            for c0, w in CHUNK_COLS:
                outs.append(jnp.pad(out[:, c0:c0 + w], ((0, 0), (0, CW - w))))
            hc_ref[...] = jnp.stack(outs, axis=0)
        else:
            o_ref[...] = out

    full = lambda shape: pl.BlockSpec(shape, lambda i: tuple(0 for _ in shape))
    in_specs = [
        pl.BlockSpec((blk, HIDDEN), lambda i: (i, 0)),
        pl.BlockSpec((NCHUNK, blk, CW), lambda i: (0, i, 0)),
        full((HIDDEN, HIDDEN)),
        full((1, HIDDEN)),
        full((HIDDEN, HIDDEN)),
        full((1, HIDDEN)),
        full((HIDDEN, 10)),
        full((10, HIDDEN)),
        full((1, HIDDEN)),
        full((1, HIDDEN)),
        full((1, 1)),
    ]
    if last:
        out_specs = pl.BlockSpec((blk, HIDDEN), lambda i: (i, 0))
        out_shape = jax.ShapeDtypeStruct((N_NODES, HIDDEN), jnp.float32)
    else:
        out_specs = [
            pl.BlockSpec((blk, HIDDEN), lambda i: (i, 0)),
            pl.BlockSpec((NCHUNK, blk, CW), lambda i: (0, i, 0)),
        ]
        out_shape = [
            jax.ShapeDtypeStruct((N_NODES, HIDDEN), jnp.float32),
            jax.ShapeDtypeStruct((NCHUNK, N_NODES, CW), jnp.float32),
        ]
    return pl.pallas_call(
        body,
        grid=(N_NODES // blk,),
        in_specs=in_specs,
        out_specs=out_specs,
        out_shape=out_shape,
    )(h, agg.reshape(NCHUNK, NP, CW), w1t, b1.reshape(1, HIDDEN), w2t,
      b2.reshape(1, HIDDEN), gm, pm, gamma.reshape(1, HIDDEN),
      beta.reshape(1, HIDDEN), epsl.reshape(1, 1))


def kernel(x, edge_index, edge_attr, W1, b1, W2, b2, eps, gamma, beta):
    src = edge_index[0]
    dst = edge_index[1]
    ac = _split_chunks(edge_attr, blk=2000).reshape(NCHUNK * N_EDGES, CW)
    hc = _split_chunks(x, blk=1000).reshape(NCHUNK * N_NODES, CW)
    h = x
    for l in range(2):
        agg = _sc_agg(hc, ac, src, dst)
        if l == 0:
            h, hc4 = _mlp_gn(h, agg, W1[l].T, b1[l], W2[l].T, b2[l], eps[l],
                             gamma[l], beta[l], last=False)
            hc = hc4.reshape(NCHUNK * N_NODES, CW)
        else:
            h = _mlp_gn(h, agg, W1[l].T, b1[l], W2[l].T, b2[l], eps[l],
                        gamma[l], beta[l], last=True)
    return h
